# baseline (speedup 1.0000x reference)
import functools
import math

import jax
import jax.numpy as jnp
from jax import lax
from jax.experimental import pallas as pl
from jax.experimental.pallas import tpu as pltpu
from jax.experimental.pallas import tpu_sc as plsc

_N = 20000
_C = 81
_IMG_W, _IMG_H = 1333, 800
_BBOX_CLIP = math.log(1000.0 / 16)

_NC, _NS, _L = 2, 16, 16
_NW = _NC * _NS
_STEP = 640
_ROWS = _STEP
_LASTBASE = 19456
_GROUPS = _ROWS // _L
_BIG = 2 ** 30

_mesh = plsc.VectorSubcoreMesh(core_axis_name="c", subcore_axis_name="s")
_mesh1 = plsc.VectorSubcoreMesh(core_axis_name="c", subcore_axis_name="s",
                                num_cores=1, num_subcores=1)


@functools.partial(
    pl.kernel,
    out_type=[
        jax.ShapeDtypeStruct((_NW * _L,), jnp.float32),
        jax.ShapeDtypeStruct((_NW * _L,), jnp.int32),
    ],
    mesh=_mesh,
    compiler_params=pltpu.CompilerParams(needs_layout_passes=False),
    scratch_types=[
        pltpu.VMEM((_C, _ROWS), jnp.float32),
        pltpu.VMEM((_L,), jnp.float32),
        pltpu.VMEM((_L,), jnp.int32),
    ],
)
def _scan_kernel(cl_hbm, out_s, out_i, buf, stage_f, stage_i):
    wid = lax.axis_index("c") * _NS + lax.axis_index("s")
    base = jnp.minimum(wid * _STEP, _LASTBASE)
    lane = lax.iota(jnp.int32, _L)

    pltpu.sync_copy(cl_hbm.at[:, pl.ds(base, _ROWS)], buf)

    def group_body(g, carry):
        bt_v, br_v, bc_v = carry
        r0 = g * _L

        def gat(c):
            return buf[c, pl.ds(r0, _L)]

        t_v = jnp.zeros((_L,), jnp.float32)
        lb_v = jnp.full((_L,), -jnp.inf, jnp.float32)
        cb_v = jnp.ones((_L,), jnp.int32)
        for k0 in range(0, _C, 16):
            k1 = min(k0 + 16, _C)
            vs = [(gat(c), c) for c in range(k0, k1)]
            cand = vs[1:] if k0 == 0 else vs
            while len(cand) > 1:
                nxt = [
                    ((lambda a, b, p: (jnp.where(p, b[0], a[0]),
                                       jnp.where(p, b[1], a[1])))
                     (cand[i], cand[i + 1], cand[i + 1][0] > cand[i][0]))
                    for i in range(0, len(cand) - 1, 2)
                ]
                if len(cand) % 2:
                    nxt.append(cand[-1])
                cand = nxt
            cmv, ccv = cand[0]
            pr2 = cmv > lb_v
            lb_new = jnp.where(pr2, cmv, lb_v)
            cb_v = jnp.where(pr2, ccv, cb_v)
            es = [jnp.exp(v - lb_new) for v, _ in vs]
            while len(es) > 1:
                tail = [es[-1]] if len(es) % 2 else []
                es = [es[i] + es[i + 1]
                      for i in range(0, len(es) - 1, 2)] + tail
            t_v = t_v * jnp.exp(lb_v - lb_new) + es[0]
            lb_v = lb_new

        row_v = base + g * _L + lane
        upd = (t_v < bt_v) & (row_v < _N)
        return (jnp.where(upd, t_v, bt_v),
                jnp.where(upd, row_v, br_v),
                jnp.where(upd, cb_v, bc_v))

    bt_v, br_v, bc_v = plsc.parallel_loop(
        0, _GROUPS, 1, unroll=4,
        carry=(jnp.full((_L,), jnp.inf, jnp.float32),
               jnp.zeros((_L,), jnp.int32),
               jnp.ones((_L,), jnp.int32)))(group_body)

    btmin = jnp.min(bt_v)
    rowc = jnp.where(bt_v == btmin, br_v, _BIG)
    brow = jnp.min(rowc)
    clsc = jnp.where(rowc == brow, bc_v, _BIG)
    bcls = jnp.min(clsc)
    sv = 1.0 / bt_v
    neginf = jnp.full((_L,), -jnp.inf, jnp.float32)
    bscore = jnp.max(jnp.where((bt_v == btmin) & (br_v == brow), sv, neginf))

    stage_f[...] = jnp.where(lane == 0, bscore, 0.0)
    stage_i[...] = jnp.where(lane == 0, brow * 128 + bcls, 0)
    pltpu.sync_copy(stage_f, out_s.at[pl.ds(wid * _L, _L)])
    pltpu.sync_copy(stage_i, out_i.at[pl.ds(wid * _L, _L)])


@functools.partial(
    pl.pallas_call,
    out_shape=[
        jax.ShapeDtypeStruct((8, 128), jnp.float32),
        jax.ShapeDtypeStruct((8, 128), jnp.float32),
        jax.ShapeDtypeStruct((8, 128), jnp.int32),
    ],
    in_specs=[
        pl.BlockSpec(memory_space=pltpu.MemorySpace.VMEM),
        pl.BlockSpec(memory_space=pltpu.MemorySpace.VMEM),
        pl.BlockSpec(memory_space=pl.ANY),
        pl.BlockSpec(memory_space=pl.ANY),
    ],
    scratch_shapes=[
        pltpu.VMEM((8, 128), jnp.float32),
        pltpu.VMEM((4, 128), jnp.float32),
        pltpu.SemaphoreType.DMA,
        pltpu.SemaphoreType.DMA,
    ],
)
def _decode_kernel(sc_ref, iv_ref, br_any, pr_any, out_b, out_s, out_c,
                   rowbuf, prbuf, sem0, sem1):
    col = lax.broadcasted_iota(jnp.int32, (4, 128), 1)
    sc = sc_ref[...].reshape(4, 128)
    iv = iv_ref[...].reshape(4, 128)
    m0 = jnp.bitwise_and(col, 15) == 0
    best = jnp.max(jnp.where(m0, sc, -jnp.inf))
    packed = jnp.min(jnp.where(m0 & (sc == best), iv, _BIG))
    row = lax.shift_right_logical(packed, 7)
    cls = jnp.bitwise_and(packed, 127)

    aligned = pl.multiple_of(jnp.bitwise_and(row, -128), 128)
    off = row - aligned
    sub0 = pl.multiple_of(jnp.bitwise_and(4 * cls, -8), 8)
    soff = 4 * cls - sub0
    cp0 = pltpu.make_async_copy(
        br_any.at[pl.ds(sub0, 8), pl.ds(aligned, 128)], rowbuf, sem0)
    cp1 = pltpu.make_async_copy(pr_any.at[:, pl.ds(aligned, 128)], prbuf, sem1)
    cp0.start()
    cp1.start()
    cp0.wait()
    cp1.wait()

    rowr = lax.broadcasted_iota(jnp.int32, (8, 128), 0)
    colr = lax.broadcasted_iota(jnp.int32, (8, 128), 1)
    rowp = lax.broadcasted_iota(jnp.int32, (4, 128), 0)
    colp = lax.broadcasted_iota(jnp.int32, (4, 128), 1)
    rv = rowbuf[...]
    pv = prbuf[...]

    def extr(k):
        return jnp.sum(jnp.where((rowr == soff + k) & (colr == off), rv, 0.0))

    def extp(k):
        return jnp.sum(jnp.where((rowp == k) & (colp == off), pv, 0.0))

    dx = extr(0) / 10.0
    dy = extr(1) / 10.0
    dw = jnp.minimum(extr(2) / 5.0, _BBOX_CLIP)
    dh = jnp.minimum(extr(3) / 5.0, _BBOX_CLIP)
    x1, y1, x2, y2 = extp(0), extp(1), extp(2), extp(3)

    w = x2 - x1 + 1.0
    h = y2 - y1 + 1.0
    cx = x1 + 0.5 * w
    cy = y1 + 0.5 * h

    pw = jnp.exp(dw) * w
    ph = jnp.exp(dh) * h
    pcx = dx * w + cx
    pcy = dy * h + cy

    def _clip(v, hi):
        return jnp.minimum(jnp.maximum(v, 0.0), hi)

    ox1 = _clip(pcx - 0.5 * pw, _IMG_W - 1.0)
    oy1 = _clip(pcy - 0.5 * ph, _IMG_H - 1.0)
    ox2 = _clip(pcx + 0.5 * pw - 1.0, _IMG_W - 1.0)
    oy2 = _clip(pcy + 0.5 * ph - 1.0, _IMG_H - 1.0)

    colo = lax.broadcasted_iota(jnp.int32, (8, 128), 1)
    out_b[...] = jnp.where(colo == 0, ox1,
                 jnp.where(colo == 1, oy1,
                 jnp.where(colo == 2, ox2,
                 jnp.where(colo == 3, oy2, 0.0))))
    out_s[...] = jnp.where(colo == 0, best, 0.0)
    out_c[...] = jnp.where(colo == 0, cls, 0)


@jax.jit
def kernel(class_logits, box_regression, proposals):
    out_s, out_i = _scan_kernel(class_logits.T)
    out_b, out_sc, out_c = _decode_kernel(out_s, out_i,
                                          box_regression.T, proposals.T)
    boxes_best = out_b[0:1, 0:4]
    max_score = out_sc[0, 0]
    cls_best = out_c[0, 0]
    return boxes_best, max_score, cls_best


# speedup vs baseline: 1.0239x; 1.0239x over previous
import functools
import math

import jax
import jax.numpy as jnp
from jax import lax
from jax.experimental import pallas as pl
from jax.experimental.pallas import tpu as pltpu
from jax.experimental.pallas import tpu_sc as plsc

_N = 20000
_C = 81
_IMG_W, _IMG_H = 1333, 800
_BBOX_CLIP = math.log(1000.0 / 16)

_NC, _NS, _L = 2, 16, 16
_NW = _NC * _NS
_STEP = 640
_ROWS = _STEP
_LASTBASE = 19456
_GROUPS = _ROWS // _L
_BIG = 2 ** 30

_mesh = plsc.VectorSubcoreMesh(core_axis_name="c", subcore_axis_name="s")
_mesh1 = plsc.VectorSubcoreMesh(core_axis_name="c", subcore_axis_name="s",
                                num_cores=1, num_subcores=1)


@functools.partial(
    pl.kernel,
    out_type=[
        jax.ShapeDtypeStruct((_NW * _L,), jnp.float32),
        jax.ShapeDtypeStruct((_NW * _L,), jnp.int32),
    ],
    mesh=_mesh,
    compiler_params=pltpu.CompilerParams(needs_layout_passes=False),
    scratch_types=[
        pltpu.VMEM((_C, _ROWS), jnp.float32),
        pltpu.VMEM((_L,), jnp.float32),
        pltpu.VMEM((_L,), jnp.int32),
    ],
)
def _scan_kernel(cl_hbm, out_s, out_i, buf, stage_f, stage_i):
    wid = lax.axis_index("c") * _NS + lax.axis_index("s")
    base = jnp.minimum(wid * _STEP, _LASTBASE)
    lane = lax.iota(jnp.int32, _L)

    pltpu.sync_copy(cl_hbm.at[:, pl.ds(base, _ROWS)], buf)

    def group_body(g, carry):
        bt_v, br_v, bc_v = carry
        r0 = g * _L

        def gat(c):
            return buf[c, pl.ds(r0, _L)]

        t_v = jnp.zeros((_L,), jnp.float32)
        lb_v = jnp.full((_L,), -jnp.inf, jnp.float32)
        cb_v = jnp.ones((_L,), jnp.int32)
        for k0 in range(0, _C, 16):
            k1 = min(k0 + 16, _C)
            vs = [(gat(c), c) for c in range(k0, k1)]
            cand = vs[1:] if k0 == 0 else vs
            while len(cand) > 1:
                nxt = [
                    ((lambda a, b, p: (jnp.where(p, b[0], a[0]),
                                       jnp.where(p, b[1], a[1])))
                     (cand[i], cand[i + 1], cand[i + 1][0] > cand[i][0]))
                    for i in range(0, len(cand) - 1, 2)
                ]
                if len(cand) % 2:
                    nxt.append(cand[-1])
                cand = nxt
            cmv, ccv = cand[0]
            pr2 = cmv > lb_v
            lb_new = jnp.where(pr2, cmv, lb_v)
            cb_v = jnp.where(pr2, ccv, cb_v)
            es = [jnp.exp(v - lb_new) for v, _ in vs]
            while len(es) > 1:
                tail = [es[-1]] if len(es) % 2 else []
                es = [es[i] + es[i + 1]
                      for i in range(0, len(es) - 1, 2)] + tail
            t_v = t_v * jnp.exp(lb_v - lb_new) + es[0]
            lb_v = lb_new

        row_v = base + g * _L + lane
        upd = (t_v < bt_v) & (row_v < _N)
        return (jnp.where(upd, t_v, bt_v),
                jnp.where(upd, row_v, br_v),
                jnp.where(upd, cb_v, bc_v))

    bt_v, br_v, bc_v = plsc.parallel_loop(
        0, _GROUPS, 1, unroll=2,
        carry=(jnp.full((_L,), jnp.inf, jnp.float32),
               jnp.zeros((_L,), jnp.int32),
               jnp.ones((_L,), jnp.int32)))(group_body)

    btmin = jnp.min(bt_v)
    rowc = jnp.where(bt_v == btmin, br_v, _BIG)
    brow = jnp.min(rowc)
    clsc = jnp.where(rowc == brow, bc_v, _BIG)
    bcls = jnp.min(clsc)
    sv = 1.0 / bt_v
    neginf = jnp.full((_L,), -jnp.inf, jnp.float32)
    bscore = jnp.max(jnp.where((bt_v == btmin) & (br_v == brow), sv, neginf))

    stage_f[...] = jnp.where(lane == 0, bscore, 0.0)
    stage_i[...] = jnp.where(lane == 0, brow * 128 + bcls, 0)
    pltpu.sync_copy(stage_f, out_s.at[pl.ds(wid * _L, _L)])
    pltpu.sync_copy(stage_i, out_i.at[pl.ds(wid * _L, _L)])


@functools.partial(
    pl.pallas_call,
    out_shape=[
        jax.ShapeDtypeStruct((8, 128), jnp.float32),
        jax.ShapeDtypeStruct((8, 128), jnp.float32),
        jax.ShapeDtypeStruct((8, 128), jnp.int32),
    ],
    in_specs=[
        pl.BlockSpec(memory_space=pltpu.MemorySpace.VMEM),
        pl.BlockSpec(memory_space=pltpu.MemorySpace.VMEM),
        pl.BlockSpec(memory_space=pl.ANY),
        pl.BlockSpec(memory_space=pl.ANY),
    ],
    scratch_shapes=[
        pltpu.VMEM((8, 128), jnp.float32),
        pltpu.VMEM((4, 128), jnp.float32),
        pltpu.SemaphoreType.DMA,
        pltpu.SemaphoreType.DMA,
    ],
)
def _decode_kernel(sc_ref, iv_ref, br_any, pr_any, out_b, out_s, out_c,
                   rowbuf, prbuf, sem0, sem1):
    col = lax.broadcasted_iota(jnp.int32, (4, 128), 1)
    sc = sc_ref[...].reshape(4, 128)
    iv = iv_ref[...].reshape(4, 128)
    m0 = jnp.bitwise_and(col, 15) == 0
    best = jnp.max(jnp.where(m0, sc, -jnp.inf))
    packed = jnp.min(jnp.where(m0 & (sc == best), iv, _BIG))
    row = lax.shift_right_logical(packed, 7)
    cls = jnp.bitwise_and(packed, 127)

    aligned = pl.multiple_of(jnp.bitwise_and(row, -128), 128)
    off = row - aligned
    sub0 = pl.multiple_of(jnp.bitwise_and(4 * cls, -8), 8)
    soff = 4 * cls - sub0
    cp0 = pltpu.make_async_copy(
        br_any.at[pl.ds(sub0, 8), pl.ds(aligned, 128)], rowbuf, sem0)
    cp1 = pltpu.make_async_copy(pr_any.at[:, pl.ds(aligned, 128)], prbuf, sem1)
    cp0.start()
    cp1.start()
    cp0.wait()
    cp1.wait()

    rowr = lax.broadcasted_iota(jnp.int32, (8, 128), 0)
    colr = lax.broadcasted_iota(jnp.int32, (8, 128), 1)
    rowp = lax.broadcasted_iota(jnp.int32, (4, 128), 0)
    colp = lax.broadcasted_iota(jnp.int32, (4, 128), 1)
    rv = rowbuf[...]
    pv = prbuf[...]

    def extr(k):
        return jnp.sum(jnp.where((rowr == soff + k) & (colr == off), rv, 0.0))

    def extp(k):
        return jnp.sum(jnp.where((rowp == k) & (colp == off), pv, 0.0))

    dx = extr(0) / 10.0
    dy = extr(1) / 10.0
    dw = jnp.minimum(extr(2) / 5.0, _BBOX_CLIP)
    dh = jnp.minimum(extr(3) / 5.0, _BBOX_CLIP)
    x1, y1, x2, y2 = extp(0), extp(1), extp(2), extp(3)

    w = x2 - x1 + 1.0
    h = y2 - y1 + 1.0
    cx = x1 + 0.5 * w
    cy = y1 + 0.5 * h

    pw = jnp.exp(dw) * w
    ph = jnp.exp(dh) * h
    pcx = dx * w + cx
    pcy = dy * h + cy

    def _clip(v, hi):
        return jnp.minimum(jnp.maximum(v, 0.0), hi)

    ox1 = _clip(pcx - 0.5 * pw, _IMG_W - 1.0)
    oy1 = _clip(pcy - 0.5 * ph, _IMG_H - 1.0)
    ox2 = _clip(pcx + 0.5 * pw - 1.0, _IMG_W - 1.0)
    oy2 = _clip(pcy + 0.5 * ph - 1.0, _IMG_H - 1.0)

    colo = lax.broadcasted_iota(jnp.int32, (8, 128), 1)
    out_b[...] = jnp.where(colo == 0, ox1,
                 jnp.where(colo == 1, oy1,
                 jnp.where(colo == 2, ox2,
                 jnp.where(colo == 3, oy2, 0.0))))
    out_s[...] = jnp.where(colo == 0, best, 0.0)
    out_c[...] = jnp.where(colo == 0, cls, 0)


@jax.jit
def kernel(class_logits, box_regression, proposals):
    out_s, out_i = _scan_kernel(class_logits.T)
    out_b, out_sc, out_c = _decode_kernel(out_s, out_i,
                                          box_regression.T, proposals.T)
    boxes_best = out_b[0:1, 0:4]
    max_score = out_sc[0, 0]
    cls_best = out_c[0, 0]
    return boxes_best, max_score, cls_best
